# baseline (speedup 1.0000x reference)
import functools

import jax
import jax.numpy as jnp
from jax import lax
from jax.experimental import pallas as pl
from jax.experimental.pallas import tpu as pltpu
from jax.experimental.pallas import tpu_sc as plsc

N = 10000
D = 128
E = 320000

NC = 2
NS = 16
NW = NC * NS
CHUNK = 128
CPT = 80
EPAD = NW * CPT * CHUNK
NPAD = 10112
RPT = NPAD // NS
BR = 400

_sc_mesh = plsc.VectorSubcoreMesh(core_axis_name="c", subcore_axis_name="s")


def _make_sc_agg(with_cnt):
    out_type = [jax.ShapeDtypeStruct((NC, NPAD, D), jnp.float32)]
    if with_cnt:
        out_type.append(jax.ShapeDtypeStruct((NC * NPAD,), jnp.float32))
        scratch = [
            pltpu.VMEM((CPT, CHUNK), jnp.int32),
            pltpu.VMEM((CHUNK,), jnp.int32),
            pltpu.VMEM((CHUNK,), jnp.int32),
            pltpu.VMEM((CHUNK, D), jnp.float32),
            pltpu.VMEM((CHUNK, D), jnp.float32),
            pltpu.VMEM((CHUNK,), jnp.float32),
            pltpu.VMEM((RPT,), jnp.float32),
            pltpu.VMEM_SHARED((NPAD, D), jnp.float32),
            pltpu.VMEM_SHARED((NPAD,), jnp.float32),
            pltpu.SemaphoreType.DMA,
            pltpu.SemaphoreType.DMA,
            pltpu.SemaphoreType.DMA,
            pltpu.SemaphoreType.DMA,
        ]
    else:
        scratch = [
            pltpu.VMEM((CPT, CHUNK), jnp.int32),
            pltpu.VMEM((CHUNK,), jnp.int32),
            pltpu.VMEM((CHUNK,), jnp.int32),
            pltpu.VMEM((CHUNK, D), jnp.float32),
            pltpu.VMEM((CHUNK, D), jnp.float32),
            pltpu.VMEM_SHARED((NPAD, D), jnp.float32),
            pltpu.SemaphoreType.DMA,
            pltpu.SemaphoreType.DMA,
            pltpu.SemaphoreType.DMA,
            pltpu.SemaphoreType.DMA,
        ]

    @functools.partial(pl.kernel, mesh=_sc_mesh, out_type=out_type,
                       scratch_types=scratch)
    def _sc_agg(y_hbm, srcs_hbm, dsts_hbm, z2d_hbm, z1d_hbm, ones_hbm,
                part_hbm, *rest):
        if with_cnt:
            cnt_hbm, idx_s, dba, dbb, ra, rb, ones_v, cbuf, acc, cntacc, \
                sga, sgb, sda, sdb = rest
        else:
            idx_s, dba, dbb, ra, rb, acc, sga, sgb, sda, sdb = rest
        c = lax.axis_index("c")
        s = lax.axis_index("s")
        w = c * NS + s
        pltpu.sync_copy(z2d_hbm.at[pl.ds(s * RPT, RPT)], acc.at[pl.ds(s * RPT, RPT)])
        pltpu.sync_copy(srcs_hbm.at[w], idx_s)
        if with_cnt:
            pltpu.sync_copy(z1d_hbm.at[pl.ds(0, RPT)], cbuf)
            pltpu.sync_copy(cbuf, cntacc.at[pl.ds(s * RPT, RPT)])
            pltpu.sync_copy(ones_hbm, ones_v)
        plsc.subcore_barrier()

        def fire(j, buf, dbuf, semg, semd):
            pltpu.async_copy(y_hbm.at[idx_s.at[j]], buf, semg)
            pltpu.async_copy(
                dsts_hbm.at[pl.ds((w * CPT + j) * CHUNK, CHUNK)], dbuf, semd)

        def drain(buf, dbuf, semg, semd):
            pltpu.make_async_copy(y_hbm.at[idx_s.at[0]], buf, semg).wait()
            pltpu.make_async_copy(dsts_hbm.at[pl.ds(0, CHUNK)], dbuf, semd).wait()
            pltpu.sync_copy(buf, acc.at[dbuf], add=True)
            if with_cnt:
                pltpu.sync_copy(ones_v, cntacc.at[dbuf], add=True)

        fire(0, ra, dba, sga, sda)

        def body(i, carry):
            j = 2 * i
            fire(j + 1, rb, dbb, sgb, sdb)
            drain(ra, dba, sga, sda)

            @pl.when(j + 2 < CPT)
            def _():
                fire(j + 2, ra, dba, sga, sda)

            drain(rb, dbb, sgb, sdb)
            return carry

        lax.fori_loop(0, CPT // 2, body, 0)

        plsc.subcore_barrier()
        pltpu.sync_copy(acc.at[pl.ds(s * RPT, RPT)], part_hbm.at[c, pl.ds(s * RPT, RPT)])
        if with_cnt:
            pltpu.sync_copy(cntacc.at[pl.ds(s * RPT, RPT)], cbuf)
            pltpu.sync_copy(cbuf, cnt_hbm.at[pl.ds(c * NPAD + s * RPT, RPT)])

    return _sc_agg


_sc_agg_cnt = _make_sc_agg(True)
_sc_agg_lean = _make_sc_agg(False)


def _lin2_body(x_ref, wl_ref, wr_ref, b_ref, y_ref, r_ref):
    xb = x_ref[...]
    y_ref[...] = jnp.dot(xb, wl_ref[...], preferred_element_type=jnp.float32)
    r_ref[...] = jnp.dot(xb, wr_ref[...], preferred_element_type=jnp.float32) + b_ref[...]


_tc_lin2 = pl.pallas_call(
    _lin2_body,
    grid=(N // BR,),
    in_specs=[
        pl.BlockSpec((BR, D), lambda i: (i, 0)),
        pl.BlockSpec((D, D), lambda i: (0, 0)),
        pl.BlockSpec((D, D), lambda i: (0, 0)),
        pl.BlockSpec((1, D), lambda i: (0, 0)),
    ],
    out_specs=[
        pl.BlockSpec((BR, D), lambda i: (i, 0)),
        pl.BlockSpec((BR, D), lambda i: (i, 0)),
    ],
    out_shape=[
        jax.ShapeDtypeStruct((N, D), jnp.float32),
        jax.ShapeDtypeStruct((N, D), jnp.float32),
    ],
)


def _mid_body(part_ref, cnt_ref, r1_ref, wl_ref, wr_ref, b_ref, y_ref, r_ref):
    seg = part_ref[0] + part_ref[1]
    cnt = jnp.maximum(cnt_ref[0] + cnt_ref[1], 1.0)
    h = jnp.maximum(seg / cnt + r1_ref[...], 0.0)
    y_ref[...] = jnp.dot(h, wl_ref[...], preferred_element_type=jnp.float32)
    r_ref[...] = jnp.dot(h, wr_ref[...], preferred_element_type=jnp.float32) + b_ref[...]


_tc_mid = pl.pallas_call(
    _mid_body,
    grid=(N // BR,),
    in_specs=[
        pl.BlockSpec((NC, BR, D), lambda i: (0, i, 0)),
        pl.BlockSpec((NC, BR, 1), lambda i: (0, i, 0)),
        pl.BlockSpec((BR, D), lambda i: (i, 0)),
        pl.BlockSpec((D, D), lambda i: (0, 0)),
        pl.BlockSpec((D, D), lambda i: (0, 0)),
        pl.BlockSpec((1, D), lambda i: (0, 0)),
    ],
    out_specs=[
        pl.BlockSpec((BR, D), lambda i: (i, 0)),
        pl.BlockSpec((BR, D), lambda i: (i, 0)),
    ],
    out_shape=[
        jax.ShapeDtypeStruct((N, D), jnp.float32),
        jax.ShapeDtypeStruct((N, D), jnp.float32),
    ],
)


def _fin_body(part_ref, cnt_ref, r2_ref, o_ref):
    seg = part_ref[0] + part_ref[1]
    cnt = jnp.maximum(cnt_ref[0] + cnt_ref[1], 1.0)
    o_ref[...] = seg / cnt + r2_ref[...]


_tc_fin = pl.pallas_call(
    _fin_body,
    grid=(N // BR,),
    in_specs=[
        pl.BlockSpec((NC, BR, D), lambda i: (0, i, 0)),
        pl.BlockSpec((NC, BR, 1), lambda i: (0, i, 0)),
        pl.BlockSpec((BR, D), lambda i: (i, 0)),
    ],
    out_specs=pl.BlockSpec((BR, D), lambda i: (i, 0)),
    out_shape=jax.ShapeDtypeStruct((N, D), jnp.float32),
)


def kernel(x, edge_index, W_l1, b_l1, W_r1, W_l2, b_l2, W_r2):
    src = edge_index[0]
    dst = edge_index[1]
    pad = EPAD - E
    srcs = jnp.concatenate([src, jnp.zeros((pad,), jnp.int32)]).reshape(NW, CPT, CHUNK)
    dsts = jnp.concatenate([dst, jnp.full((pad,), N, jnp.int32)])
    z2d = jnp.zeros((NPAD, D), jnp.float32)
    z1d = jnp.zeros((NPAD,), jnp.float32)
    ones = jnp.ones((CHUNK,), jnp.float32)
    b1 = b_l1.reshape(1, D)
    b2 = b_l2.reshape(1, D)

    y1, r1 = _tc_lin2(x, W_l1, W_r1, b1)
    part1, cnt = _sc_agg_cnt(y1, srcs, dsts, z2d, z1d, ones)
    cnt3 = cnt.reshape(NC, NPAD, 1)
    y2, r2 = _tc_mid(part1, cnt3, r1, W_l2, W_r2, b2)
    (part2,) = _sc_agg_lean(y2, srcs, dsts, z2d, z1d, ones)
    return _tc_fin(part2, cnt3, r2)


# speedup vs baseline: 1.2561x; 1.2561x over previous
import functools

import jax
import jax.numpy as jnp
from jax import lax
from jax.experimental import pallas as pl
from jax.experimental.pallas import tpu as pltpu
from jax.experimental.pallas import tpu_sc as plsc

N = 10000
D = 128
E = 320000

NC = 2
NS = 16
NW = NC * NS
CHUNK = 128
CPT = 79
EPAD = NW * CPT * CHUNK
NPAD = 10112
RPT = NPAD // NS
BR = 400

_sc_mesh = plsc.VectorSubcoreMesh(core_axis_name="c", subcore_axis_name="s")


def _make_sc_agg(with_cnt):
    out_type = [jax.ShapeDtypeStruct((NC, NPAD, D), jnp.float32)]
    if with_cnt:
        out_type.append(jax.ShapeDtypeStruct((NC * NPAD,), jnp.float32))
        scratch = [
            pltpu.VMEM((CPT, CHUNK), jnp.int32),
            pltpu.VMEM((CPT, CHUNK), jnp.int32),
            pltpu.VMEM((CHUNK, D), jnp.float32),
            pltpu.VMEM((CHUNK,), jnp.float32),
            pltpu.VMEM((RPT,), jnp.float32),
            pltpu.VMEM_SHARED((NPAD, D), jnp.float32),
            pltpu.VMEM_SHARED((NPAD,), jnp.float32),
            pltpu.SemaphoreType.DMA,
            pltpu.SemaphoreType.DMA,
        ]
    else:
        scratch = [
            pltpu.VMEM((CPT, CHUNK), jnp.int32),
            pltpu.VMEM((CPT, CHUNK), jnp.int32),
            pltpu.VMEM((CHUNK, D), jnp.float32),
            pltpu.VMEM_SHARED((NPAD, D), jnp.float32),
            pltpu.SemaphoreType.DMA,
            pltpu.SemaphoreType.DMA,
        ]

    @functools.partial(pl.kernel, mesh=_sc_mesh, out_type=out_type,
                       scratch_types=scratch)
    def _sc_agg(y_hbm, srcs_hbm, dsts_hbm, z2d_hbm, z1d_hbm, ones_hbm,
                part_hbm, *rest):
        if with_cnt:
            cnt_hbm, idx_s, idx_d, rows, ones_v, cbuf, acc, cntacc, sem0, sem1 = rest
        else:
            idx_s, idx_d, rows, acc, sem0, sem1 = rest
        c = lax.axis_index("c")
        s = lax.axis_index("s")
        w = c * NS + s
        pltpu.sync_copy(z2d_hbm.at[pl.ds(s * RPT, RPT)], acc.at[pl.ds(s * RPT, RPT)])
        pltpu.sync_copy(srcs_hbm.at[w], idx_s)
        pltpu.sync_copy(dsts_hbm.at[w], idx_d)
        if with_cnt:
            pltpu.sync_copy(z1d_hbm.at[pl.ds(0, RPT)], cbuf)
            pltpu.sync_copy(cbuf, cntacc.at[pl.ds(s * RPT, RPT)])
            pltpu.sync_copy(ones_hbm, ones_v)
        plsc.subcore_barrier()

        half = CHUNK // 2

        def body(i, carry):
            cp0 = pltpu.async_copy(
                y_hbm.at[c].at[idx_s.at[i, pl.ds(0, half)]], rows.at[pl.ds(0, half)], sem0)
            cp1 = pltpu.async_copy(
                y_hbm.at[c].at[idx_s.at[i, pl.ds(half, half)]], rows.at[pl.ds(half, half)], sem1)
            cp0.wait()
            cp1.wait()
            pltpu.sync_copy(rows, acc.at[idx_d.at[i]], add=True)
            if with_cnt:
                pltpu.sync_copy(ones_v, cntacc.at[idx_d.at[i]], add=True)
            return carry

        lax.fori_loop(0, CPT, body, 0)

        plsc.subcore_barrier()
        pltpu.sync_copy(acc.at[pl.ds(s * RPT, RPT)], part_hbm.at[c, pl.ds(s * RPT, RPT)])
        if with_cnt:
            pltpu.sync_copy(cntacc.at[pl.ds(s * RPT, RPT)], cbuf)
            pltpu.sync_copy(cbuf, cnt_hbm.at[pl.ds(c * NPAD + s * RPT, RPT)])

    return _sc_agg


_sc_agg_cnt = _make_sc_agg(True)
_sc_agg_lean = _make_sc_agg(False)


def _lin2_body(x_ref, wl_ref, wr_ref, b_ref, y_ref, r_ref):
    xb = x_ref[...]
    y = jnp.dot(xb, wl_ref[...], preferred_element_type=jnp.float32)
    y_ref[0] = y
    y_ref[1] = y
    r_ref[...] = jnp.dot(xb, wr_ref[...], preferred_element_type=jnp.float32) + b_ref[...]


_tc_lin2 = pl.pallas_call(
    _lin2_body,
    grid=(N // BR,),
    in_specs=[
        pl.BlockSpec((BR, D), lambda i: (i, 0)),
        pl.BlockSpec((D, D), lambda i: (0, 0)),
        pl.BlockSpec((D, D), lambda i: (0, 0)),
        pl.BlockSpec((1, D), lambda i: (0, 0)),
    ],
    out_specs=[
        pl.BlockSpec((NC, BR, D), lambda i: (0, i, 0)),
        pl.BlockSpec((BR, D), lambda i: (i, 0)),
    ],
    out_shape=[
        jax.ShapeDtypeStruct((NC, N, D), jnp.float32),
        jax.ShapeDtypeStruct((N, D), jnp.float32),
    ],
)


def _mid_body(part_ref, cnt_ref, r1_ref, wl_ref, wr_ref, b_ref, y_ref, r_ref):
    seg = part_ref[0] + part_ref[1]
    cnt = jnp.maximum(cnt_ref[0] + cnt_ref[1], 1.0)
    h = jnp.maximum(seg / cnt + r1_ref[...], 0.0)
    y = jnp.dot(h, wl_ref[...], preferred_element_type=jnp.float32)
    y_ref[0] = y
    y_ref[1] = y
    r_ref[...] = jnp.dot(h, wr_ref[...], preferred_element_type=jnp.float32) + b_ref[...]


_tc_mid = pl.pallas_call(
    _mid_body,
    grid=(N // BR,),
    in_specs=[
        pl.BlockSpec((NC, BR, D), lambda i: (0, i, 0)),
        pl.BlockSpec((NC, BR, 1), lambda i: (0, i, 0)),
        pl.BlockSpec((BR, D), lambda i: (i, 0)),
        pl.BlockSpec((D, D), lambda i: (0, 0)),
        pl.BlockSpec((D, D), lambda i: (0, 0)),
        pl.BlockSpec((1, D), lambda i: (0, 0)),
    ],
    out_specs=[
        pl.BlockSpec((NC, BR, D), lambda i: (0, i, 0)),
        pl.BlockSpec((BR, D), lambda i: (i, 0)),
    ],
    out_shape=[
        jax.ShapeDtypeStruct((NC, N, D), jnp.float32),
        jax.ShapeDtypeStruct((N, D), jnp.float32),
    ],
)


def _fin_body(part_ref, cnt_ref, r2_ref, o_ref):
    seg = part_ref[0] + part_ref[1]
    cnt = jnp.maximum(cnt_ref[0] + cnt_ref[1], 1.0)
    o_ref[...] = seg / cnt + r2_ref[...]


_tc_fin = pl.pallas_call(
    _fin_body,
    grid=(N // BR,),
    in_specs=[
        pl.BlockSpec((NC, BR, D), lambda i: (0, i, 0)),
        pl.BlockSpec((NC, BR, 1), lambda i: (0, i, 0)),
        pl.BlockSpec((BR, D), lambda i: (i, 0)),
    ],
    out_specs=pl.BlockSpec((BR, D), lambda i: (i, 0)),
    out_shape=jax.ShapeDtypeStruct((N, D), jnp.float32),
)


def kernel(x, edge_index, W_l1, b_l1, W_r1, W_l2, b_l2, W_r2):
    src = edge_index[0]
    dst = edge_index[1]
    pad = EPAD - E
    srcs = jnp.concatenate([src, jnp.zeros((pad,), jnp.int32)]).reshape(NW, CPT, CHUNK)
    dsts = jnp.concatenate([dst, jnp.full((pad,), N, jnp.int32)]).reshape(NW, CPT, CHUNK)
    z2d = jnp.zeros((NPAD, D), jnp.float32)
    z1d = jnp.zeros((NPAD,), jnp.float32)
    ones = jnp.ones((CHUNK,), jnp.float32)
    b1 = b_l1.reshape(1, D)
    b2 = b_l2.reshape(1, D)

    y1, r1 = _tc_lin2(x, W_l1, W_r1, b1)
    part1, cnt = _sc_agg_cnt(y1, srcs, dsts, z2d, z1d, ones)
    cnt3 = cnt.reshape(NC, NPAD, 1)
    y2, r2 = _tc_mid(part1, cnt3, r1, W_l2, W_r2, b2)
    (part2,) = _sc_agg_lean(y2, srcs, dsts, z2d, z1d, ones)
    return _tc_fin(part2, cnt3, r2)


# speedup vs baseline: 1.3527x; 1.0769x over previous
import functools

import jax
import jax.numpy as jnp
from jax import lax
from jax.experimental import pallas as pl
from jax.experimental.pallas import tpu as pltpu
from jax.experimental.pallas import tpu_sc as plsc

N = 10000
D = 128
E = 320000

NC = 2
NS = 16
NW = NC * NS
CHUNK = 128
CPT = 79
EPAD = NW * CPT * CHUNK
NPAD = 10112
RPT = NPAD // NS
BR = 400

_sc_mesh = plsc.VectorSubcoreMesh(core_axis_name="c", subcore_axis_name="s")


def _make_sc_agg(with_cnt):
    out_type = [jax.ShapeDtypeStruct((NC, NPAD, D), jnp.float32)]
    if with_cnt:
        out_type.append(jax.ShapeDtypeStruct((NC * NPAD,), jnp.float32))
        scratch = [
            pltpu.VMEM((CPT, CHUNK), jnp.int32),
            pltpu.VMEM((CPT, CHUNK), jnp.int32),
            pltpu.VMEM((CHUNK, D), jnp.float32),
            pltpu.VMEM((CHUNK,), jnp.float32),
            pltpu.VMEM((RPT,), jnp.float32),
            pltpu.VMEM_SHARED((NPAD, D), jnp.float32),
            pltpu.VMEM_SHARED((NPAD,), jnp.float32),
            pltpu.SemaphoreType.DMA,
            pltpu.SemaphoreType.DMA,
        ]
    else:
        scratch = [
            pltpu.VMEM((CPT, CHUNK), jnp.int32),
            pltpu.VMEM((CPT, CHUNK), jnp.int32),
            pltpu.VMEM((CHUNK, D), jnp.float32),
            pltpu.VMEM_SHARED((NPAD, D), jnp.float32),
            pltpu.SemaphoreType.DMA,
            pltpu.SemaphoreType.DMA,
        ]

    @functools.partial(pl.kernel, mesh=_sc_mesh, out_type=out_type,
                       scratch_types=scratch)
    def _sc_agg(y_hbm, srcs_hbm, dsts_hbm, z2d_hbm, z1d_hbm, ones_hbm,
                part_hbm, *rest):
        if with_cnt:
            cnt_hbm, idx_s, idx_d, rows, ones_v, cbuf, acc, cntacc, sem0, sem1 = rest
        else:
            idx_s, idx_d, rows, acc, sem0, sem1 = rest
        c = lax.axis_index("c")
        s = lax.axis_index("s")
        w = c * NS + s
        pltpu.sync_copy(z2d_hbm.at[pl.ds(s * RPT, RPT)], acc.at[pl.ds(s * RPT, RPT)])
        pltpu.sync_copy(srcs_hbm.at[w], idx_s)
        pltpu.sync_copy(dsts_hbm.at[w], idx_d)
        if with_cnt:
            pltpu.sync_copy(z1d_hbm.at[pl.ds(0, RPT)], cbuf)
            pltpu.sync_copy(cbuf, cntacc.at[pl.ds(s * RPT, RPT)])
            pltpu.sync_copy(ones_hbm, ones_v)
        plsc.subcore_barrier()

        half = CHUNK // 2

        def body(i, carry):
            cp0 = pltpu.async_copy(
                y_hbm.at[idx_s.at[i, pl.ds(0, half)]], rows.at[pl.ds(0, half)], sem0)
            cp1 = pltpu.async_copy(
                y_hbm.at[idx_s.at[i, pl.ds(half, half)]], rows.at[pl.ds(half, half)], sem1)
            cp0.wait()
            cp1.wait()
            pltpu.sync_copy(rows, acc.at[idx_d.at[i]], add=True)
            if with_cnt:
                pltpu.sync_copy(ones_v, cntacc.at[idx_d.at[i]], add=True)
            return carry

        lax.fori_loop(0, CPT, body, 0)

        plsc.subcore_barrier()
        pltpu.sync_copy(acc.at[pl.ds(s * RPT, RPT)], part_hbm.at[c, pl.ds(s * RPT, RPT)])
        if with_cnt:
            pltpu.sync_copy(cntacc.at[pl.ds(s * RPT, RPT)], cbuf)
            pltpu.sync_copy(cbuf, cnt_hbm.at[pl.ds(c * NPAD + s * RPT, RPT)])

    return _sc_agg


_sc_agg_cnt = _make_sc_agg(True)
_sc_agg_lean = _make_sc_agg(False)


def _lin2_body(x_ref, wl_ref, wr_ref, b_ref, y_ref, r_ref):
    xb = x_ref[...]
    y_ref[...] = jnp.dot(xb, wl_ref[...], preferred_element_type=jnp.float32)
    r_ref[...] = jnp.dot(xb, wr_ref[...], preferred_element_type=jnp.float32) + b_ref[...]


_tc_lin2 = pl.pallas_call(
    _lin2_body,
    grid=(N // BR,),
    in_specs=[
        pl.BlockSpec((BR, D), lambda i: (i, 0)),
        pl.BlockSpec((D, D), lambda i: (0, 0)),
        pl.BlockSpec((D, D), lambda i: (0, 0)),
        pl.BlockSpec((1, D), lambda i: (0, 0)),
    ],
    out_specs=[
        pl.BlockSpec((BR, D), lambda i: (i, 0)),
        pl.BlockSpec((BR, D), lambda i: (i, 0)),
    ],
    out_shape=[
        jax.ShapeDtypeStruct((N, D), jnp.float32),
        jax.ShapeDtypeStruct((N, D), jnp.float32),
    ],
)


def _mid_body(part_ref, cnt_ref, r1_ref, wl_ref, wr_ref, b_ref, y_ref, r_ref):
    seg = part_ref[0] + part_ref[1]
    cnt = jnp.maximum(cnt_ref[0] + cnt_ref[1], 1.0)
    h = jnp.maximum(seg / cnt + r1_ref[...], 0.0)
    y_ref[...] = jnp.dot(h, wl_ref[...], preferred_element_type=jnp.float32)
    r_ref[...] = jnp.dot(h, wr_ref[...], preferred_element_type=jnp.float32) + b_ref[...]


_tc_mid = pl.pallas_call(
    _mid_body,
    grid=(N // BR,),
    in_specs=[
        pl.BlockSpec((NC, BR, D), lambda i: (0, i, 0)),
        pl.BlockSpec((NC, BR, 1), lambda i: (0, i, 0)),
        pl.BlockSpec((BR, D), lambda i: (i, 0)),
        pl.BlockSpec((D, D), lambda i: (0, 0)),
        pl.BlockSpec((D, D), lambda i: (0, 0)),
        pl.BlockSpec((1, D), lambda i: (0, 0)),
    ],
    out_specs=[
        pl.BlockSpec((BR, D), lambda i: (i, 0)),
        pl.BlockSpec((BR, D), lambda i: (i, 0)),
    ],
    out_shape=[
        jax.ShapeDtypeStruct((N, D), jnp.float32),
        jax.ShapeDtypeStruct((N, D), jnp.float32),
    ],
)


def _fin_body(part_ref, cnt_ref, r2_ref, o_ref):
    seg = part_ref[0] + part_ref[1]
    cnt = jnp.maximum(cnt_ref[0] + cnt_ref[1], 1.0)
    o_ref[...] = seg / cnt + r2_ref[...]


_tc_fin = pl.pallas_call(
    _fin_body,
    grid=(N // BR,),
    in_specs=[
        pl.BlockSpec((NC, BR, D), lambda i: (0, i, 0)),
        pl.BlockSpec((NC, BR, 1), lambda i: (0, i, 0)),
        pl.BlockSpec((BR, D), lambda i: (i, 0)),
    ],
    out_specs=pl.BlockSpec((BR, D), lambda i: (i, 0)),
    out_shape=jax.ShapeDtypeStruct((N, D), jnp.float32),
)


def kernel(x, edge_index, W_l1, b_l1, W_r1, W_l2, b_l2, W_r2):
    src = edge_index[0]
    dst = edge_index[1]
    pad = EPAD - E
    srcs = jnp.concatenate([src, jnp.zeros((pad,), jnp.int32)]).reshape(NW, CPT, CHUNK)
    dsts = jnp.concatenate([dst, jnp.full((pad,), N, jnp.int32)]).reshape(NW, CPT, CHUNK)
    z2d = jnp.zeros((NPAD, D), jnp.float32)
    z1d = jnp.zeros((NPAD,), jnp.float32)
    ones = jnp.ones((CHUNK,), jnp.float32)
    b1 = b_l1.reshape(1, D)
    b2 = b_l2.reshape(1, D)

    y1, r1 = _tc_lin2(x, W_l1, W_r1, b1)
    part1, cnt = _sc_agg_cnt(y1, srcs, dsts, z2d, z1d, ones)
    cnt3 = cnt.reshape(NC, NPAD, 1)
    y2, r2 = _tc_mid(part1, cnt3, r1, W_l2, W_r2, b2)
    (part2,) = _sc_agg_lean(y2, srcs, dsts, z2d, z1d, ones)
    return _tc_fin(part2, cnt3, r2)


# speedup vs baseline: 1.3767x; 1.0178x over previous
import functools

import jax
import jax.numpy as jnp
from jax import lax
from jax.experimental import pallas as pl
from jax.experimental.pallas import tpu as pltpu
from jax.experimental.pallas import tpu_sc as plsc

N = 10000
D = 128
E = 320000

NC = 2
NS = 16
NW = NC * NS
CHUNK = 128
CPT = 79
EPAD = NW * CPT * CHUNK
NPAD = 10112
RPT = NPAD // NS
BR = 400

_sc_mesh = plsc.VectorSubcoreMesh(core_axis_name="c", subcore_axis_name="s")


def _make_sc_agg(with_cnt):
    out_type = [jax.ShapeDtypeStruct((NC, NPAD, D), jnp.float32)]
    if with_cnt:
        out_type.append(jax.ShapeDtypeStruct((NC * NPAD,), jnp.float32))
        scratch = [
            pltpu.VMEM((CPT, CHUNK), jnp.int32),
            pltpu.VMEM((CPT, CHUNK), jnp.int32),
            pltpu.VMEM((CHUNK, D), jnp.float32),
            pltpu.VMEM((CHUNK,), jnp.float32),
            pltpu.VMEM((RPT,), jnp.float32),
            pltpu.VMEM_SHARED((NPAD, D), jnp.float32),
            pltpu.VMEM_SHARED((NPAD,), jnp.float32),
            pltpu.SemaphoreType.DMA,
            pltpu.SemaphoreType.DMA,
        ]
    else:
        scratch = [
            pltpu.VMEM((CPT, CHUNK), jnp.int32),
            pltpu.VMEM((CPT, CHUNK), jnp.int32),
            pltpu.VMEM((CHUNK, D), jnp.float32),
            pltpu.VMEM_SHARED((NPAD, D), jnp.float32),
            pltpu.SemaphoreType.DMA,
            pltpu.SemaphoreType.DMA,
        ]

    @functools.partial(pl.kernel, mesh=_sc_mesh, out_type=out_type,
                       scratch_types=scratch,
                       compiler_params=pltpu.CompilerParams(use_tc_tiling_on_sc=False))
    def _sc_agg(y_hbm, srcs_hbm, dsts_hbm, z2d_hbm, z1d_hbm, ones_hbm,
                part_hbm, *rest):
        if with_cnt:
            cnt_hbm, idx_s, idx_d, rows, ones_v, cbuf, acc, cntacc, sem0, sem1 = rest
        else:
            idx_s, idx_d, rows, acc, sem0, sem1 = rest
        c = lax.axis_index("c")
        s = lax.axis_index("s")
        w = c * NS + s
        pltpu.sync_copy(z2d_hbm.at[pl.ds(s * RPT, RPT)], acc.at[pl.ds(s * RPT, RPT)])
        pltpu.sync_copy(srcs_hbm.at[w], idx_s)
        pltpu.sync_copy(dsts_hbm.at[w], idx_d)
        if with_cnt:
            pltpu.sync_copy(z1d_hbm.at[pl.ds(0, RPT)], cbuf)
            pltpu.sync_copy(cbuf, cntacc.at[pl.ds(s * RPT, RPT)])
            pltpu.sync_copy(ones_hbm, ones_v)
        plsc.subcore_barrier()

        half = CHUNK // 2

        def body(i, carry):
            cp0 = pltpu.async_copy(
                y_hbm.at[idx_s.at[i, pl.ds(0, half)]], rows.at[pl.ds(0, half)], sem0)
            cp1 = pltpu.async_copy(
                y_hbm.at[idx_s.at[i, pl.ds(half, half)]], rows.at[pl.ds(half, half)], sem1)
            cp0.wait()
            cp1.wait()
            pltpu.sync_copy(rows, acc.at[idx_d.at[i]], add=True)
            if with_cnt:
                pltpu.sync_copy(ones_v, cntacc.at[idx_d.at[i]], add=True)
            return carry

        lax.fori_loop(0, CPT, body, 0)

        plsc.subcore_barrier()
        pltpu.sync_copy(acc.at[pl.ds(s * RPT, RPT)], part_hbm.at[c, pl.ds(s * RPT, RPT)])
        if with_cnt:
            pltpu.sync_copy(cntacc.at[pl.ds(s * RPT, RPT)], cbuf)
            pltpu.sync_copy(cbuf, cnt_hbm.at[pl.ds(c * NPAD + s * RPT, RPT)])

    return _sc_agg


_sc_agg_cnt = _make_sc_agg(True)
_sc_agg_lean = _make_sc_agg(False)


def _lin2_body(x_ref, wl_ref, wr_ref, b_ref, y_ref, r_ref):
    xb = x_ref[...]
    y_ref[...] = jnp.dot(xb, wl_ref[...], preferred_element_type=jnp.float32)
    r_ref[...] = jnp.dot(xb, wr_ref[...], preferred_element_type=jnp.float32) + b_ref[...]


_tc_lin2 = pl.pallas_call(
    _lin2_body,
    grid=(N // BR,),
    in_specs=[
        pl.BlockSpec((BR, D), lambda i: (i, 0)),
        pl.BlockSpec((D, D), lambda i: (0, 0)),
        pl.BlockSpec((D, D), lambda i: (0, 0)),
        pl.BlockSpec((1, D), lambda i: (0, 0)),
    ],
    out_specs=[
        pl.BlockSpec((BR, D), lambda i: (i, 0)),
        pl.BlockSpec((BR, D), lambda i: (i, 0)),
    ],
    out_shape=[
        jax.ShapeDtypeStruct((N, D), jnp.float32),
        jax.ShapeDtypeStruct((N, D), jnp.float32),
    ],
)


def _mid_body(part_ref, cnt_ref, r1_ref, wl_ref, wr_ref, b_ref, y_ref, r_ref):
    seg = part_ref[0] + part_ref[1]
    cnt = jnp.maximum(cnt_ref[0] + cnt_ref[1], 1.0)
    h = jnp.maximum(seg / cnt + r1_ref[...], 0.0)
    y_ref[...] = jnp.dot(h, wl_ref[...], preferred_element_type=jnp.float32)
    r_ref[...] = jnp.dot(h, wr_ref[...], preferred_element_type=jnp.float32) + b_ref[...]


_tc_mid = pl.pallas_call(
    _mid_body,
    grid=(N // BR,),
    in_specs=[
        pl.BlockSpec((NC, BR, D), lambda i: (0, i, 0)),
        pl.BlockSpec((NC, BR, 1), lambda i: (0, i, 0)),
        pl.BlockSpec((BR, D), lambda i: (i, 0)),
        pl.BlockSpec((D, D), lambda i: (0, 0)),
        pl.BlockSpec((D, D), lambda i: (0, 0)),
        pl.BlockSpec((1, D), lambda i: (0, 0)),
    ],
    out_specs=[
        pl.BlockSpec((BR, D), lambda i: (i, 0)),
        pl.BlockSpec((BR, D), lambda i: (i, 0)),
    ],
    out_shape=[
        jax.ShapeDtypeStruct((N, D), jnp.float32),
        jax.ShapeDtypeStruct((N, D), jnp.float32),
    ],
)


def _fin_body(part_ref, cnt_ref, r2_ref, o_ref):
    seg = part_ref[0] + part_ref[1]
    cnt = jnp.maximum(cnt_ref[0] + cnt_ref[1], 1.0)
    o_ref[...] = seg / cnt + r2_ref[...]


_tc_fin = pl.pallas_call(
    _fin_body,
    grid=(N // BR,),
    in_specs=[
        pl.BlockSpec((NC, BR, D), lambda i: (0, i, 0)),
        pl.BlockSpec((NC, BR, 1), lambda i: (0, i, 0)),
        pl.BlockSpec((BR, D), lambda i: (i, 0)),
    ],
    out_specs=pl.BlockSpec((BR, D), lambda i: (i, 0)),
    out_shape=jax.ShapeDtypeStruct((N, D), jnp.float32),
)


def kernel(x, edge_index, W_l1, b_l1, W_r1, W_l2, b_l2, W_r2):
    src = edge_index[0]
    dst = edge_index[1]
    pad = EPAD - E
    srcs = jnp.concatenate([src, jnp.zeros((pad,), jnp.int32)]).reshape(NW, CPT, CHUNK)
    dsts = jnp.concatenate([dst, jnp.full((pad,), N, jnp.int32)]).reshape(NW, CPT, CHUNK)
    z2d = jnp.zeros((NPAD, D), jnp.float32)
    z1d = jnp.zeros((NPAD,), jnp.float32)
    ones = jnp.ones((CHUNK,), jnp.float32)
    b1 = b_l1.reshape(1, D)
    b2 = b_l2.reshape(1, D)

    y1, r1 = _tc_lin2(x, W_l1, W_r1, b1)
    part1, cnt = _sc_agg_cnt(y1, srcs, dsts, z2d, z1d, ones)
    cnt3 = cnt.reshape(NC, NPAD, 1)
    y2, r2 = _tc_mid(part1, cnt3, r1, W_l2, W_r2, b2)
    (part2,) = _sc_agg_lean(y2, srcs, dsts, z2d, z1d, ones)
    return _tc_fin(part2, cnt3, r2)
